# baseline (speedup 1.0000x reference)
import functools

import jax
import jax.numpy as jnp
from jax import lax
from jax.experimental import pallas as pl
from jax.experimental.pallas import tpu as pltpu
from jax.experimental.pallas import tpu_sc as plsc

_NUM_TOKENS = 32
_DIM = 768
_BATCH = 16384

_INFO = plsc.get_sparse_core_info()
_NC = _INFO.num_cores
_NS = _INFO.num_subcores
_NW = _NC * _NS
_BPW = _BATCH // _NW
_CHUNK = 64
_NCHUNKS = _BPW // _CHUNK


def _body(tokens_hbm, idx_hbm, out_hbm, idx_v, rows0, rows1, g0, g1, o0, o1):
    wid = lax.axis_index("s") * _NC + lax.axis_index("c")
    base = wid * _BPW
    pltpu.sync_copy(idx_hbm.at[pl.ds(base, _BPW)], idx_v)

    rows = (rows0, rows1)
    gsem = (g0, g1)
    osem = (o0, o1)
    gathers = [None, None]
    scatters = [None, None]

    gathers[0] = pltpu.async_copy(
        tokens_hbm.at[idx_v.at[pl.ds(0, _CHUNK)]], rows[0], gsem[0])
    for c in range(_NCHUNKS):
        b = c & 1
        nb = (c + 1) & 1
        if c + 1 < _NCHUNKS:
            if scatters[nb] is not None:
                scatters[nb].wait()
            gathers[nb] = pltpu.async_copy(
                tokens_hbm.at[idx_v.at[pl.ds((c + 1) * _CHUNK, _CHUNK)]],
                rows[nb], gsem[nb])
        gathers[b].wait()
        scatters[b] = pltpu.async_copy(
            rows[b], out_hbm.at[pl.ds(base + c * _CHUNK, _CHUNK)], osem[b])
    scatters[(_NCHUNKS - 2) & 1].wait()
    scatters[(_NCHUNKS - 1) & 1].wait()


_lookup = functools.partial(
    pl.kernel,
    out_type=jax.ShapeDtypeStruct((_BATCH, _DIM), jnp.float32),
    mesh=plsc.VectorSubcoreMesh(core_axis_name="c", subcore_axis_name="s"),
    scratch_types=[
        pltpu.VMEM((_BPW,), jnp.int32),
        pltpu.VMEM((_CHUNK, _DIM), jnp.float32),
        pltpu.VMEM((_CHUNK, _DIM), jnp.float32),
        pltpu.SemaphoreType.DMA,
        pltpu.SemaphoreType.DMA,
        pltpu.SemaphoreType.DMA,
        pltpu.SemaphoreType.DMA,
    ],
)(_body)


@jax.jit
def kernel(tokens, indices):
    return _lookup(tokens, indices)
